# baseline (speedup 1.0000x reference)
import functools

import jax
import jax.numpy as jnp
from jax import lax
from jax.experimental import pallas as pl
from jax.experimental.pallas import tpu as pltpu
from jax.experimental.pallas import tpu_sc as plsc

_K = 1024
_D = 64
_BETA = 0.25
_HW = 1024
_B = 16
_N = _B * _HW

_BB = 2

_NC = 2
_NS = 16
_NW = _NC * _NS
_ROWS_PER_W = _N // _NW


def _vq_block(z_ref, w_ref, idx_ref, loss_ref):
    i = pl.program_id(0)
    w = w_ref[...]
    w2 = jnp.sum(w ** 2, axis=1, keepdims=True)

    @pl.when(i == 0)
    def _init():
        loss_ref[...] = jnp.zeros_like(loss_ref)

    w2x = w + w
    for j in range(_BB):
        zd = z_ref[j]
        z2 = jnp.sum(zd ** 2, axis=0, keepdims=True)
        s2 = jax.lax.dot_general(
            w2x, zd, (((1,), (0,)), ((), ())),
            preferred_element_type=jnp.float32)
        d2 = (z2 + w2) - s2
        m = jnp.min(d2, axis=0, keepdims=True)
        iota = jax.lax.broadcasted_iota(jnp.int32, d2.shape, 0)
        cand = jnp.where(d2 == m, iota, _K)
        idx = jnp.min(cand, axis=0, keepdims=True)
        idx_ref[j] = jnp.broadcast_to(idx, (8, _HW))
        loss_ref[...] += jnp.sum(m) * ((1.0 + _BETA) / (_N * _D))


_SC_MESH = plsc.VectorSubcoreMesh(core_axis_name="c", subcore_axis_name="s")


def _make_sc_gather(n):
    rows_per_w = n // _NW

    @functools.partial(
        pl.kernel,
        out_type=jax.ShapeDtypeStruct((n, 128), jnp.float32),
        mesh=_SC_MESH,
        scratch_types=[
            pltpu.VMEM((rows_per_w,), jnp.int32),
            pltpu.VMEM((rows_per_w, 128), jnp.float32),
            pltpu.SemaphoreType.DMA,
        ],
    )
    def _sc_gather(w_hbm, idx_hbm, out_hbm, idx_v, rows_v, sem):
        wid = lax.axis_index("s") * _NC + lax.axis_index("c")
        base = wid * rows_per_w
        pltpu.sync_copy(idx_hbm.at[pl.ds(base, rows_per_w)], idx_v)
        pltpu.async_copy(w_hbm.at[idx_v], rows_v, sem).wait()
        pltpu.sync_copy(rows_v, out_hbm.at[pl.ds(base, rows_per_w)])

    return _sc_gather


_sc_gather_half = _make_sc_gather(_N // 2)


def _vq_argmin_half(zr, W, half):
    off = half * (_B // 2 // _BB)
    return pl.pallas_call(
        _vq_block,
        grid=(_B // 2 // _BB,),
        in_specs=[
            pl.BlockSpec((_BB, _D, _HW), lambda i: (i + off, 0, 0)),
            pl.BlockSpec((_K, _D), lambda i: (0, 0)),
        ],
        out_specs=[
            pl.BlockSpec((_BB, 8, _HW), lambda i: (i, 0, 0)),
            pl.BlockSpec((1, 1), lambda i: (0, 0)),
        ],
        out_shape=[
            jax.ShapeDtypeStruct((_B // 2, 8, _HW), jnp.int32),
            jax.ShapeDtypeStruct((1, 1), jnp.float32),
        ],
    )(zr, W)


def kernel(z, W):
    zr = z.reshape(_B, _D, _HW)
    W128 = jnp.pad(W, ((0, 0), (0, 128 - _D)))
    idx_a, loss_a = _vq_argmin_half(zr, W, 0)
    idx_b, loss_b = _vq_argmin_half(zr, W, 1)
    rows_a = _sc_gather_half(W128, idx_a[:, 0, :].reshape(-1))
    rows_b = _sc_gather_half(W128, idx_b[:, 0, :].reshape(-1))
    zq_rows = jnp.concatenate([rows_a, rows_b], axis=0)
    out = jnp.transpose(zq_rows.reshape(_B, 32, 32, 128),
                        (0, 3, 1, 2))[:, :_D]
    return out, loss_a[0, 0] + loss_b[0, 0]


# speedup vs baseline: 1.2974x; 1.2974x over previous
import jax
import jax.numpy as jnp
from jax.experimental import pallas as pl
from jax.experimental.pallas import tpu as pltpu

_K = 1024
_D = 64
_BETA = 0.25
_HW = 1024
_B = 16
_N = _B * _HW


_BB = 2


def _vq_block(z_ref, w_ref, wt_ref, out_ref, loss_ref):
    i = pl.program_id(0)
    w = w_ref[...]
    wt = wt_ref[...]
    w2 = jnp.sum(w ** 2, axis=1, keepdims=True)
    wt_hi = wt.astype(jnp.bfloat16)
    wt_lo = (wt - wt_hi.astype(jnp.float32)).astype(jnp.bfloat16)
    gdims = (((1,), (0,)), ((), ()))

    @pl.when(i == 0)
    def _init():
        loss_ref[...] = jnp.zeros_like(loss_ref)

    w2x = w + w
    for j in range(_BB):
        zd = z_ref[j]
        z2 = jnp.sum(zd ** 2, axis=0, keepdims=True)
        s2 = jax.lax.dot_general(
            w2x, zd, (((1,), (0,)), ((), ())),
            preferred_element_type=jnp.float32)
        d2 = (z2 + w2) - s2
        m = jnp.min(d2, axis=0, keepdims=True)
        iota = jax.lax.broadcasted_iota(jnp.int32, d2.shape, 0)
        cand = jnp.where(d2 == m, iota, _K)
        idx = jnp.min(cand, axis=0, keepdims=True)
        onehot = (cand == idx).astype(jnp.bfloat16)
        zq = (jax.lax.dot_general(wt_hi, onehot, gdims,
                                  preferred_element_type=jnp.float32)
              + jax.lax.dot_general(wt_lo, onehot, gdims,
                                    preferred_element_type=jnp.float32))
        out_ref[j] = zd + (zq - zd)
        loss_ref[...] += jnp.sum(m) * ((1.0 + _BETA) / (_N * _D))


def kernel(z, W):
    zr = z.reshape(_B, _D, _HW)
    Wt = W.T
    zq3, loss = pl.pallas_call(
        _vq_block,
        grid=(_B // _BB,),
        in_specs=[
            pl.BlockSpec((_BB, _D, _HW), lambda i: (i, 0, 0)),
            pl.BlockSpec((_K, _D), lambda i: (0, 0)),
            pl.BlockSpec((_D, _K), lambda i: (0, 0)),
        ],
        out_specs=[
            pl.BlockSpec((_BB, _D, _HW), lambda i: (i, 0, 0)),
            pl.BlockSpec((1, 1), lambda i: (0, 0)),
        ],
        out_shape=[
            jax.ShapeDtypeStruct((_B, _D, _HW), jnp.float32),
            jax.ShapeDtypeStruct((1, 1), jnp.float32),
        ],
    )(zr, W, Wt)
    return zq3.reshape(z.shape), loss[0, 0]
